# Initial kernel scaffold; baseline (speedup 1.0000x reference)
import jax
import jax.numpy as jnp
from jax.experimental import pallas as pl


def kernel(pc, node_a, node_b, first_pn_out, second_pn_out, node_a_features, node_b_features, global_feature, img_s16_feature_map, img_s32_feature_map, img_global_feature, params, node_a_min_k_idx):
    raise NotImplementedError("write your pallas kernel here")


# speedup vs baseline: 15.7273x; 15.7273x over previous
import jax
import jax.numpy as jnp
from jax.experimental import pallas as pl
from jax.experimental.pallas import tpu as pltpu

_F32 = jnp.float32
_NB = 1024


def _dot(a, b, precision=None):
    return jax.lax.dot_general(a, b, (((1,), (0,)), ((), ())),
                               preferred_element_type=_F32, precision=precision)


def _dot_t1(a, b):
    return jax.lax.dot_general(a, b, (((1,), (1,)), ((), ())),
                               preferred_element_type=_F32)


def _top3_weights(d, m):
    iota = jax.lax.broadcasted_iota(jnp.int32, d.shape, 1)
    dm = d
    ds, ohs = [], []
    for _ in range(3):
        vmin = jnp.min(dm, axis=1, keepdims=True)
        first = jnp.min(jnp.where(dm == vmin, iota, m), axis=1, keepdims=True)
        oh = iota == first
        ds.append(vmin)
        ohs.append(oh)
        dm = jnp.where(oh, jnp.inf, dm)
    s_tot = ds[0] + ds[1] + ds[2]
    out = jnp.zeros(d.shape, _F32)
    for k in range(3):
        out = out + jnp.where(ohs[k], 1.0 - ds[k] / s_tot, 0.0)
    return out


def _pairwise_dist(pts_t, nodes):
    d2 = jnp.zeros((pts_t.shape[0], nodes.shape[1]), _F32)
    for c in range(3):
        diff = pts_t[:, c:c + 1] - nodes[c:c + 1, :]
        d2 = d2 + diff * diff
    return jnp.sqrt(d2)


def _stage1_kernel(nbf_ref, naf_ref, g_ref, ig_ref, s16_ref, s32_ref,
                   na_t_ref, nb_ref,
                   W1a_ref, W1b_ref, b1_ref, W2_ref, b2_ref,
                   V1a_ref, V1b_ref, V1c_ref, V1d_ref, c1_ref,
                   V2_ref, c2_ref, V3_ref, c3_ref,
                   A1a_ref, A1b_ref, a1_ref, A2_ref, a2_ref,
                   U1a_ref, U1b_ref, U1c_ref, u1_ref,
                   U2_ref, u2_ref, U3_ref, u3_ref,
                   up_nb_ref, up_na_ref):
    nbf = nbf_ref[0]
    naf = naf_ref[0]
    g = g_ref[0]
    ig = ig_ref[0]
    s16 = s16_ref[0]
    s32 = s32_ref[0]
    na_t = na_t_ref[0]
    nb = nb_ref[0]

    relu = jax.nn.relu
    t = relu(_dot(W1a_ref[...], nbf) + _dot(W1b_ref[...], ig) + b1_ref[...])
    nb_att = _dot(W2_ref[...], t) + b2_ref[...]
    nb_w = _dot(s32_ref[0], nb_att) * (1.0 / 80.0)
    q = relu(_dot(V1a_ref[...], nbf) + _dot(V1b_ref[...], g)
             + _dot(V1c_ref[...], nb_w) + _dot(V1d_ref[...], ig) + c1_ref[...])
    q = relu(_dot(V2_ref[...], q) + c2_ref[...])
    up_nb = _dot(V3_ref[...], q) + c3_ref[...]
    up_nb_ref[0] = up_nb
    r = relu(_dot(A1a_ref[...], naf) + _dot(A1b_ref[...], ig) + a1_ref[...])
    na_att = _dot(A2_ref[...], r) + a2_ref[...]
    na_w = _dot(s16, na_att) * (1.0 / 320.0)
    d = _pairwise_dist(na_t, nb)
    s_sel = _top3_weights(d, 64)
    interp_ab = _dot_t1(up_nb, s_sel)
    z = relu(_dot(U1a_ref[...], naf) + _dot(U1b_ref[...], interp_ab)
             + _dot(U1c_ref[...], na_w) + u1_ref[...])
    z = relu(_dot(U2_ref[...], z) + u2_ref[...])
    up_na_ref[0] = _dot(U3_ref[...], z) + u3_ref[...]


def _stage2_kernel(p_ref, ii_ref, f1_ref, f2_ref, nb_ref, na_t_ref,
                   upnb_t_ref, upna_t_ref,
                   P1a_ref, P1b_ref, P1c_ref, P1d_ref, p1_ref,
                   P2_ref, p2_ref, P3_ref, p3_ref,
                   coarse_ref, fine_ref):
    p = p_ref[0]
    nb = nb_ref[0]
    na_t = na_t_ref[0]
    relu = jax.nn.relu

    d = _pairwise_dist(p, nb)
    s_sel = _top3_weights(d, 64)
    interp_pb = _dot(s_sel, upnb_t_ref[0])

    ii = ii_ref[0]
    iota = jax.lax.broadcasted_iota(jnp.int32, (p.shape[0], 256), 1)
    ohs, ds = [], []
    for k in range(3):
        oh = (iota == ii[:, k:k + 1]).astype(_F32)
        coords = _dot(oh, na_t_ref[0], precision=jax.lax.Precision.HIGHEST)
        diff = p - coords
        ds.append(jnp.sqrt(jnp.sum(diff * diff, axis=1, keepdims=True)))
        ohs.append(oh)
    s_tot = ds[0] + ds[1] + ds[2]
    s_a = jnp.zeros_like(ohs[0])
    for k in range(3):
        s_a = s_a + (1.0 - ds[k] / s_tot) * ohs[k]
    interp_pa = _dot(s_a, upna_t_ref[0])

    h = relu(_dot(interp_pa, P1a_ref[...]) + _dot(interp_pb, P1b_ref[...])
             + _dot(f1_ref[0], P1c_ref[...]) + _dot(f2_ref[0], P1d_ref[...])
             + p1_ref[...])
    h = relu(_dot(h, P2_ref[...]) + p2_ref[...])
    o_t = _dot_t1(P3_ref[...], h) + p3_ref[...]
    coarse_ref[0] = o_t[0:2, :]
    fine_ref[0] = o_t[2:82, :]


def kernel(pc, node_a, node_b, first_pn_out, second_pn_out, node_a_features,
           node_b_features, global_feature, img_s16_feature_map,
           img_s32_feature_map, img_global_feature, params, node_a_min_k_idx):
    B, N = pc.shape[0], pc.shape[2]
    Ma, Mb = node_a.shape[2], node_b.shape[2]
    f32 = _F32

    s16 = img_s16_feature_map.reshape(B, img_s16_feature_map.shape[1], -1)
    s32 = img_s32_feature_map.reshape(B, img_s32_feature_map.shape[1], -1)
    ig = img_global_feature.reshape(B, img_global_feature.shape[1], 1)
    na_t = node_a.transpose(0, 2, 1)
    pc_t = pc.transpose(0, 2, 1)
    f1_t = first_pn_out.transpose(0, 2, 1)
    f2_t = second_pn_out.transpose(0, 2, 1)
    idx = node_a_min_k_idx.astype(jnp.int32)

    col = lambda b: b.reshape(-1, 1)
    row = lambda b: b.reshape(1, -1)

    (W1, b1), (W2, b2) = params['nb_att']
    (V1, c1), (V2, c2), (V3, c3) = params['nb_pn']
    (A1, a1), (A2, a2) = params['na_att']
    (U1, u1), (U2, u2), (U3, u3) = params['na_pn']
    (P1, q1), (P2, q2), (P3, q3) = params['pp_pn']

    w_s1 = [W1[:, :256], W1[:, 256:], col(b1), W2, col(b2),
            V1[:, :256], V1[:, 256:768], V1[:, 768:1280], V1[:, 1280:], col(c1),
            V2, col(c2), V3, col(c3),
            A1[:, :64], A1[:, 64:], col(a1), A2, col(a2),
            U1[:, :64], U1[:, 64:576], U1[:, 576:], col(u1),
            U2, col(u2), U3, col(u3)]

    bspec = lambda *s: pl.BlockSpec((1,) + s, lambda b: (b, 0, 0))
    wspec = lambda w: pl.BlockSpec(w.shape, lambda b: (0,) * w.ndim)

    up_nb, up_na = pl.pallas_call(
        _stage1_kernel,
        grid=(B,),
        in_specs=[bspec(256, Mb), bspec(64, Ma), bspec(512, 1), bspec(512, 1),
                  bspec(256, 320), bspec(512, 80), bspec(Ma, 3), bspec(3, Mb)]
                 + [wspec(w) for w in w_s1],
        out_specs=[bspec(512, Mb), bspec(128, Ma)],
        out_shape=[jax.ShapeDtypeStruct((B, 512, Mb), f32),
                   jax.ShapeDtypeStruct((B, 128, Ma), f32)],
    )(node_b_features, node_a_features, global_feature, ig, s16, s32,
      na_t, node_b, *w_s1)

    upnb_t = up_nb.transpose(0, 2, 1)
    upna_t = up_na.transpose(0, 2, 1)

    w_s2 = [P1[:, :128].T, P1[:, 128:640].T, P1[:, 640:672].T, P1[:, 672:].T,
            row(q1), P2.T, row(q2), P3, col(q3)]

    bspec2 = lambda *s: pl.BlockSpec((1,) + s, lambda b, i: (b, i, 0))
    rep2 = lambda *s: pl.BlockSpec((1,) + s, lambda b, i: (b, 0, 0))
    wspec2 = lambda w: pl.BlockSpec(w.shape, lambda b, i: (0,) * w.ndim)

    coarse, fine = pl.pallas_call(
        _stage2_kernel,
        grid=(B, N // _NB),
        in_specs=[bspec2(_NB, 3), bspec2(_NB, 3), bspec2(_NB, 32),
                  bspec2(_NB, 64), rep2(3, Mb), rep2(Ma, 3),
                  rep2(Mb, 512), rep2(Ma, 128)]
                 + [wspec2(w) for w in w_s2],
        out_specs=[pl.BlockSpec((1, 2, _NB), lambda b, i: (b, 0, i)),
                   pl.BlockSpec((1, 80, _NB), lambda b, i: (b, 0, i))],
        out_shape=[jax.ShapeDtypeStruct((B, 2, N), f32),
                   jax.ShapeDtypeStruct((B, 80, N), f32)],
        compiler_params=pltpu.CompilerParams(
            dimension_semantics=("parallel", "parallel")),
    )(pc_t, idx, f1_t, f2_t, node_b, na_t, upnb_t, upna_t, *w_s2)

    return (coarse, fine)


# speedup vs baseline: 33.0816x; 2.1035x over previous
import jax
import jax.numpy as jnp
from jax.experimental import pallas as pl
from jax.experimental.pallas import tpu as pltpu

_F32 = jnp.float32
_NB = 2048
_IMAX = (1 << 31) - 1


def _dot(a, b, precision=None):
    return jax.lax.dot_general(a, b, (((1,), (0,)), ((), ())),
                               preferred_element_type=_F32, precision=precision)


def _dot_t1(a, b):
    return jax.lax.dot_general(a, b, (((1,), (1,)), ((), ())),
                               preferred_element_type=_F32)


def _top3_weights_cols(d):
    iota = jax.lax.broadcasted_iota(jnp.int32, d.shape, 0)
    di = (jax.lax.bitcast_convert_type(d, jnp.int32) & ~63) | iota
    for _ in range(3):
        m = jnp.min(di, axis=0, keepdims=True)
        di = jnp.where(di == m, _IMAX, di)
    sel = di == _IMAX
    s_sum = jnp.sum(jnp.where(sel, d, 0.0), axis=0, keepdims=True)
    return jnp.where(sel, 1.0 - d * (1.0 / s_sum), 0.0)


def _dist_cols(nodes_t, pts):
    d2 = None
    for c in range(3):
        diff = nodes_t[:, c:c + 1] - pts[c:c + 1, :]
        d2 = diff * diff if d2 is None else d2 + diff * diff
    return jnp.sqrt(d2)


def _stage1_kernel(nbf_ref, naf_ref, g_ref, ig_ref, s16_ref, s32_ref,
                   na_ref, nb_t_ref,
                   W1a_ref, W1b_ref, b1_ref, W2_ref, b2_ref,
                   V1a_ref, V1b_ref, V1c_ref, V1d_ref, c1_ref,
                   V2_ref, c2_ref, V3_ref, c3_ref,
                   A1a_ref, A1b_ref, a1_ref, A2_ref, a2_ref,
                   U1a_ref, U1b_ref, U1c_ref, u1_ref,
                   U2_ref, u2_ref, U3_ref, u3_ref,
                   up_nb_ref, up_na_ref):
    nbf = nbf_ref[0]
    naf = naf_ref[0]
    g = g_ref[0]
    ig = ig_ref[0]
    s16 = s16_ref[0]
    na = na_ref[0]
    nb_t = nb_t_ref[0]

    relu = jax.nn.relu
    t = relu(_dot(W1a_ref[...], nbf) + _dot(W1b_ref[...], ig) + b1_ref[...])
    nb_att = _dot(W2_ref[...], t) + b2_ref[...]
    nb_w = _dot(s32_ref[0], nb_att) * (1.0 / 80.0)
    q = relu(_dot(V1a_ref[...], nbf) + _dot(V1b_ref[...], g)
             + _dot(V1c_ref[...], nb_w) + _dot(V1d_ref[...], ig) + c1_ref[...])
    q = relu(_dot(V2_ref[...], q) + c2_ref[...])
    up_nb = _dot(V3_ref[...], q) + c3_ref[...]
    up_nb_ref[0] = up_nb
    r = relu(_dot(A1a_ref[...], naf) + _dot(A1b_ref[...], ig) + a1_ref[...])
    na_att = _dot(A2_ref[...], r) + a2_ref[...]
    na_w = _dot(s16, na_att) * (1.0 / 320.0)
    d = _dist_cols(nb_t, na)
    s_sel = _top3_weights_cols(d)
    interp_ab = _dot(up_nb, s_sel)
    z = relu(_dot(U1a_ref[...], naf) + _dot(U1b_ref[...], interp_ab)
             + _dot(U1c_ref[...], na_w) + u1_ref[...])
    z = relu(_dot(U2_ref[...], z) + u2_ref[...])
    up_na_ref[0] = _dot(U3_ref[...], z) + u3_ref[...]


def _stage2_kernel(p_ref, ii_ref, f1_ref, f2_ref, nb_t_ref, na_ref,
                   upnb_ref, upna_ref,
                   P1a_ref, P1b_ref, P1c_ref, P1d_ref, p1_ref,
                   P2_ref, p2_ref, P3_ref, p3_ref,
                   coarse_ref, fine_ref):
    p = p_ref[0]
    na = na_ref[0]
    relu = jax.nn.relu

    d = _dist_cols(nb_t_ref[0], p)
    s_sel = _top3_weights_cols(d)
    interp_pb = _dot(upnb_ref[0], s_sel)

    ii = ii_ref[0]
    iota = jax.lax.broadcasted_iota(jnp.int32, (256, p.shape[1]), 0)
    ohs, ds = [], []
    for k in range(3):
        oh = (iota == ii[k:k + 1, :]).astype(_F32)
        coords = _dot(na, oh, precision=jax.lax.Precision.HIGHEST)
        diff = p - coords
        ds.append(jnp.sqrt(jnp.sum(diff * diff, axis=0, keepdims=True)))
        ohs.append(oh)
    rs = 1.0 / (ds[0] + ds[1] + ds[2])
    s_a = (1.0 - ds[0] * rs) * ohs[0]
    for k in range(1, 3):
        s_a = s_a + (1.0 - ds[k] * rs) * ohs[k]
    interp_pa = _dot(upna_ref[0], s_a)

    h = relu(_dot(P1a_ref[...], interp_pa) + _dot(P1b_ref[...], interp_pb)
             + _dot(P1c_ref[...], f1_ref[0]) + _dot(P1d_ref[...], f2_ref[0])
             + p1_ref[...])
    h = relu(_dot(P2_ref[...], h) + p2_ref[...])
    o = _dot(P3_ref[...], h) + p3_ref[...]
    coarse_ref[0] = o[0:2, :]
    fine_ref[0] = o[2:82, :]


def kernel(pc, node_a, node_b, first_pn_out, second_pn_out, node_a_features,
           node_b_features, global_feature, img_s16_feature_map,
           img_s32_feature_map, img_global_feature, params, node_a_min_k_idx):
    B, N = pc.shape[0], pc.shape[2]
    Ma, Mb = node_a.shape[2], node_b.shape[2]
    f32 = _F32

    s16 = img_s16_feature_map.reshape(B, img_s16_feature_map.shape[1], -1)
    s32 = img_s32_feature_map.reshape(B, img_s32_feature_map.shape[1], -1)
    ig = img_global_feature.reshape(B, img_global_feature.shape[1], 1)
    nb_t = node_b.transpose(0, 2, 1)
    ii_t = node_a_min_k_idx.astype(jnp.int32).transpose(0, 2, 1)

    col = lambda b: b.reshape(-1, 1)

    (W1, b1), (W2, b2) = params['nb_att']
    (V1, c1), (V2, c2), (V3, c3) = params['nb_pn']
    (A1, a1), (A2, a2) = params['na_att']
    (U1, u1), (U2, u2), (U3, u3) = params['na_pn']
    (P1, q1), (P2, q2), (P3, q3) = params['pp_pn']

    w_s1 = [W1[:, :256], W1[:, 256:], col(b1), W2, col(b2),
            V1[:, :256], V1[:, 256:768], V1[:, 768:1280], V1[:, 1280:], col(c1),
            V2, col(c2), V3, col(c3),
            A1[:, :64], A1[:, 64:], col(a1), A2, col(a2),
            U1[:, :64], U1[:, 64:576], U1[:, 576:], col(u1),
            U2, col(u2), U3, col(u3)]

    bspec = lambda *s: pl.BlockSpec((1,) + s, lambda b: (b, 0, 0))
    wspec = lambda w: pl.BlockSpec(w.shape, lambda b: (0,) * w.ndim)

    up_nb, up_na = pl.pallas_call(
        _stage1_kernel,
        grid=(B,),
        in_specs=[bspec(256, Mb), bspec(64, Ma), bspec(512, 1), bspec(512, 1),
                  bspec(256, 320), bspec(512, 80), bspec(3, Ma), bspec(Mb, 3)]
                 + [wspec(w) for w in w_s1],
        out_specs=[bspec(512, Mb), bspec(128, Ma)],
        out_shape=[jax.ShapeDtypeStruct((B, 512, Mb), f32),
                   jax.ShapeDtypeStruct((B, 128, Ma), f32)],
    )(node_b_features, node_a_features, global_feature, ig, s16, s32,
      node_a, nb_t, *w_s1)

    w_s2 = [P1[:, :128], P1[:, 128:640], P1[:, 640:672], P1[:, 672:],
            col(q1), P2, col(q2), P3, col(q3)]

    bspec2 = lambda *s: pl.BlockSpec((1,) + s, lambda b, i: (b, 0, i))
    rep2 = lambda *s: pl.BlockSpec((1,) + s, lambda b, i: (b, 0, 0))
    wspec2 = lambda w: pl.BlockSpec(w.shape, lambda b, i: (0,) * w.ndim)

    coarse, fine = pl.pallas_call(
        _stage2_kernel,
        grid=(B, N // _NB),
        in_specs=[bspec2(3, _NB), bspec2(3, _NB), bspec2(32, _NB),
                  bspec2(64, _NB), rep2(Mb, 3), rep2(3, Ma),
                  rep2(512, Mb), rep2(128, Ma)]
                 + [wspec2(w) for w in w_s2],
        out_specs=[pl.BlockSpec((1, 2, _NB), lambda b, i: (b, 0, i)),
                   pl.BlockSpec((1, 80, _NB), lambda b, i: (b, 0, i))],
        out_shape=[jax.ShapeDtypeStruct((B, 2, N), f32),
                   jax.ShapeDtypeStruct((B, 80, N), f32)],
        compiler_params=pltpu.CompilerParams(
            dimension_semantics=("parallel", "parallel")),
    )(pc, ii_t, first_pn_out, second_pn_out, nb_t, node_a, up_nb, up_na, *w_s2)

    return (coarse, fine)


# speedup vs baseline: 44.8495x; 1.3557x over previous
import jax
import jax.numpy as jnp
from jax.experimental import pallas as pl
from jax.experimental.pallas import tpu as pltpu

_F32 = jnp.float32
_NB = 2048
_IMAX = (1 << 31) - 1


def _dot(a, b, precision=None):
    return jax.lax.dot_general(a, b, (((1,), (0,)), ((), ())),
                               preferred_element_type=_F32, precision=precision)


def _dot_t1(a, b):
    return jax.lax.dot_general(a, b, (((1,), (1,)), ((), ())),
                               preferred_element_type=_F32)


def _top3_weights_cols(d):
    iota = jax.lax.broadcasted_iota(jnp.int32, d.shape, 0)
    di = (jax.lax.bitcast_convert_type(d, jnp.int32) & ~63) | iota
    for _ in range(3):
        m = jnp.min(di, axis=0, keepdims=True)
        di = jnp.where(di == m, _IMAX, di)
    sel = di == _IMAX
    s_sum = jnp.sum(jnp.where(sel, d, 0.0), axis=0, keepdims=True)
    return jnp.where(sel, 1.0 - d * (1.0 / s_sum), 0.0)


def _dist_cols(nodes_t, pts):
    d2 = None
    for c in range(3):
        diff = nodes_t[:, c:c + 1] - pts[c:c + 1, :]
        d2 = diff * diff if d2 is None else d2 + diff * diff
    return jnp.sqrt(d2)


def _stage1_kernel(nbf_ref, naf_ref, g_ref, ig_ref, s16_ref, s32_ref,
                   na_ref, nb_t_ref,
                   W1a_ref, W1b_ref, b1_ref, W2_ref, b2_ref,
                   V1a_ref, V1b_ref, V1c_ref, V1d_ref, c1_ref,
                   V2_ref, c2_ref, V3_ref, c3_ref,
                   A1a_ref, A1b_ref, a1_ref, A2_ref, a2_ref,
                   U1a_ref, U1b_ref, U1c_ref, u1_ref,
                   U2_ref, u2_ref, U3_ref, u3_ref,
                   P1a_ref, P1b_ref,
                   mb_ref, ma_ref):
    nbf = nbf_ref[0]
    naf = naf_ref[0]
    g = g_ref[0]
    ig = ig_ref[0]
    s16 = s16_ref[0]
    na = na_ref[0]
    nb_t = nb_t_ref[0]

    relu = jax.nn.relu
    t = relu(_dot(W1a_ref[...], nbf) + _dot(W1b_ref[...], ig) + b1_ref[...])
    nb_att = _dot(W2_ref[...], t) + b2_ref[...]
    nb_w = _dot(s32_ref[0], nb_att) * (1.0 / 80.0)
    q = relu(_dot(V1a_ref[...], nbf) + _dot(V1b_ref[...], g)
             + _dot(V1c_ref[...], nb_w) + _dot(V1d_ref[...], ig) + c1_ref[...])
    q = relu(_dot(V2_ref[...], q) + c2_ref[...])
    up_nb = _dot(V3_ref[...], q) + c3_ref[...]
    mb_ref[0] = _dot(P1b_ref[...], up_nb)
    r = relu(_dot(A1a_ref[...], naf) + _dot(A1b_ref[...], ig) + a1_ref[...])
    na_att = _dot(A2_ref[...], r) + a2_ref[...]
    na_w = _dot(s16, na_att) * (1.0 / 320.0)
    d = _dist_cols(nb_t, na)
    s_sel = _top3_weights_cols(d)
    interp_ab = _dot(up_nb, s_sel)
    z = relu(_dot(U1a_ref[...], naf) + _dot(U1b_ref[...], interp_ab)
             + _dot(U1c_ref[...], na_w) + u1_ref[...])
    z = relu(_dot(U2_ref[...], z) + u2_ref[...])
    up_na = _dot(U3_ref[...], z) + u3_ref[...]
    ma_ref[0] = _dot(P1a_ref[...], up_na)


def _stage2_kernel(p_ref, ii_ref, f1_ref, f2_ref, nb_t_ref, na_ref,
                   mb_ref, ma_ref,
                   P1c_ref, P1d_ref, p1_ref,
                   P2_ref, p2_ref, P3_ref, p3_ref,
                   coarse_ref, fine_ref):
    p = p_ref[0]
    na = na_ref[0]
    relu = jax.nn.relu

    d = _dist_cols(nb_t_ref[0], p)
    s_sel = _top3_weights_cols(d)

    ii = ii_ref[0]
    iota = jax.lax.broadcasted_iota(jnp.int32, (256, p.shape[1]), 0)
    ohs, ds = [], []
    for k in range(3):
        oh = (iota == ii[k:k + 1, :]).astype(_F32)
        coords = _dot(na, oh)
        diff = p - coords
        ds.append(jnp.sqrt(jnp.sum(diff * diff, axis=0, keepdims=True)))
        ohs.append(oh)
    rs = 1.0 / (ds[0] + ds[1] + ds[2])
    s_a = (1.0 - ds[0] * rs) * ohs[0]
    for k in range(1, 3):
        s_a = s_a + (1.0 - ds[k] * rs) * ohs[k]

    h = relu(_dot(mb_ref[0], s_sel) + _dot(ma_ref[0], s_a)
             + _dot(P1c_ref[...], f1_ref[0]) + _dot(P1d_ref[...], f2_ref[0])
             + p1_ref[...])
    h = relu(_dot(P2_ref[...], h) + p2_ref[...])
    o = _dot(P3_ref[...], h) + p3_ref[...]
    coarse_ref[0] = o[0:2, :]
    fine_ref[0] = o[2:82, :]


def kernel(pc, node_a, node_b, first_pn_out, second_pn_out, node_a_features,
           node_b_features, global_feature, img_s16_feature_map,
           img_s32_feature_map, img_global_feature, params, node_a_min_k_idx):
    B, N = pc.shape[0], pc.shape[2]
    Ma, Mb = node_a.shape[2], node_b.shape[2]
    f32 = _F32

    s16 = img_s16_feature_map.reshape(B, img_s16_feature_map.shape[1], -1)
    s32 = img_s32_feature_map.reshape(B, img_s32_feature_map.shape[1], -1)
    ig = img_global_feature.reshape(B, img_global_feature.shape[1], 1)
    nb_t = node_b.transpose(0, 2, 1)
    ii_t = node_a_min_k_idx.astype(jnp.int32).transpose(0, 2, 1)

    col = lambda b: b.reshape(-1, 1)

    (W1, b1), (W2, b2) = params['nb_att']
    (V1, c1), (V2, c2), (V3, c3) = params['nb_pn']
    (A1, a1), (A2, a2) = params['na_att']
    (U1, u1), (U2, u2), (U3, u3) = params['na_pn']
    (P1, q1), (P2, q2), (P3, q3) = params['pp_pn']

    w_s1 = [W1[:, :256], W1[:, 256:], col(b1), W2, col(b2),
            V1[:, :256], V1[:, 256:768], V1[:, 768:1280], V1[:, 1280:], col(c1),
            V2, col(c2), V3, col(c3),
            A1[:, :64], A1[:, 64:], col(a1), A2, col(a2),
            U1[:, :64], U1[:, 64:576], U1[:, 576:], col(u1),
            U2, col(u2), U3, col(u3),
            P1[:, :128], P1[:, 128:640]]

    bspec = lambda *s: pl.BlockSpec((1,) + s, lambda b: (b, 0, 0))
    wspec = lambda w: pl.BlockSpec(w.shape, lambda b: (0,) * w.ndim)

    mb, ma = pl.pallas_call(
        _stage1_kernel,
        grid=(B,),
        in_specs=[bspec(256, Mb), bspec(64, Ma), bspec(512, 1), bspec(512, 1),
                  bspec(256, 320), bspec(512, 80), bspec(3, Ma), bspec(Mb, 3)]
                 + [wspec(w) for w in w_s1],
        out_specs=[bspec(256, Mb), bspec(256, Ma)],
        out_shape=[jax.ShapeDtypeStruct((B, 256, Mb), f32),
                   jax.ShapeDtypeStruct((B, 256, Ma), f32)],
    )(node_b_features, node_a_features, global_feature, ig, s16, s32,
      node_a, nb_t, *w_s1)

    w_s2 = [P1[:, 640:672], P1[:, 672:], col(q1), P2, col(q2), P3, col(q3)]

    bspec2 = lambda *s: pl.BlockSpec((1,) + s, lambda b, i: (b, 0, i))
    rep2 = lambda *s: pl.BlockSpec((1,) + s, lambda b, i: (b, 0, 0))
    wspec2 = lambda w: pl.BlockSpec(w.shape, lambda b, i: (0,) * w.ndim)

    coarse, fine = pl.pallas_call(
        _stage2_kernel,
        grid=(B, N // _NB),
        in_specs=[bspec2(3, _NB), bspec2(3, _NB), bspec2(32, _NB),
                  bspec2(64, _NB), rep2(Mb, 3), rep2(3, Ma),
                  rep2(256, Mb), rep2(256, Ma)]
                 + [wspec2(w) for w in w_s2],
        out_specs=[pl.BlockSpec((1, 2, _NB), lambda b, i: (b, 0, i)),
                   pl.BlockSpec((1, 80, _NB), lambda b, i: (b, 0, i))],
        out_shape=[jax.ShapeDtypeStruct((B, 2, N), f32),
                   jax.ShapeDtypeStruct((B, 80, N), f32)],
        compiler_params=pltpu.CompilerParams(
            dimension_semantics=("parallel", "parallel")),
    )(pc, ii_t, first_pn_out, second_pn_out, nb_t, node_a, mb, ma, *w_s2)

    return (coarse, fine)


# speedup vs baseline: 50.3082x; 1.1217x over previous
import jax
import jax.numpy as jnp
from jax.experimental import pallas as pl
from jax.experimental.pallas import tpu as pltpu

_F32 = jnp.float32
_NB = 2048
_IMAX = (1 << 31) - 1


def _dot(a, b, precision=None):
    return jax.lax.dot_general(a, b, (((1,), (0,)), ((), ())),
                               preferred_element_type=_F32, precision=precision)


def _dot_t1(a, b):
    return jax.lax.dot_general(a, b, (((1,), (1,)), ((), ())),
                               preferred_element_type=_F32)


def _top3_weights_cols(d):
    iota = jax.lax.broadcasted_iota(jnp.int32, d.shape, 0)
    di = (jax.lax.bitcast_convert_type(d, jnp.int32) & ~63) | iota
    for _ in range(3):
        m = jnp.min(di, axis=0, keepdims=True)
        di = jnp.where(di == m, _IMAX, di)
    sel = di == _IMAX
    s_sum = jnp.sum(jnp.where(sel, d, 0.0), axis=0, keepdims=True)
    return jnp.where(sel, 1.0 - d * (1.0 / s_sum), 0.0)


def _dist_cols(nodes_t, pts):
    d2 = None
    for c in range(3):
        diff = nodes_t[:, c:c + 1] - pts[c:c + 1, :]
        d2 = diff * diff if d2 is None else d2 + diff * diff
    return jnp.sqrt(d2)


def _stage1_kernel(nbf_ref, naf_ref, g_ref, ig_ref, s16_ref, s32_ref,
                   na_ref, nb_t_ref,
                   W1_ref, b1_ref, W2_ref, b2_ref,
                   V1_ref, c1_ref, V2_ref, c2_ref, V3_ref, c3_ref,
                   A1_ref, a1_ref, A2_ref, a2_ref,
                   U1_ref, u1_ref, U2_ref, u2_ref, U3_ref, u3_ref,
                   P1_ref,
                   mb_ref, ma_ref):
    nbf = nbf_ref[0]
    naf = naf_ref[0]
    g = g_ref[0]
    ig = ig_ref[0]
    s16 = s16_ref[0]
    na = na_ref[0]
    nb_t = nb_t_ref[0]

    relu = jax.nn.relu
    t = relu(_dot(W1_ref[:, :256], nbf) + _dot(W1_ref[:, 256:], ig)
             + b1_ref[...])
    nb_att = _dot(W2_ref[...], t) + b2_ref[...]
    nb_w = _dot(s32_ref[0], nb_att) * (1.0 / 80.0)
    q = relu(_dot(V1_ref[:, :256], nbf) + _dot(V1_ref[:, 256:768], g)
             + _dot(V1_ref[:, 768:1280], nb_w) + _dot(V1_ref[:, 1280:], ig)
             + c1_ref[...])
    q = relu(_dot(V2_ref[...], q) + c2_ref[...])
    up_nb = _dot(V3_ref[...], q) + c3_ref[...]
    mb_ref[0] = _dot(P1_ref[:, 128:640], up_nb)
    r = relu(_dot(A1_ref[:, :64], naf) + _dot(A1_ref[:, 64:], ig)
             + a1_ref[...])
    na_att = _dot(A2_ref[...], r) + a2_ref[...]
    na_w = _dot(s16, na_att) * (1.0 / 320.0)
    d = _dist_cols(nb_t, na)
    s_sel = _top3_weights_cols(d)
    interp_ab = _dot(up_nb, s_sel)
    z = relu(_dot(U1_ref[:, :64], naf) + _dot(U1_ref[:, 64:576], interp_ab)
             + _dot(U1_ref[:, 576:], na_w) + u1_ref[...])
    z = relu(_dot(U2_ref[...], z) + u2_ref[...])
    up_na = _dot(U3_ref[...], z) + u3_ref[...]
    ma_ref[0] = _dot(P1_ref[:, :128], up_na)


def _stage2_kernel(p_ref, ii_ref, f1_ref, f2_ref, nb_t_ref, na_ref,
                   mb_ref, ma_ref,
                   P1_ref, p1_ref,
                   P2_ref, p2_ref, P3_ref, p3_ref,
                   coarse_ref, fine_ref):
    p = p_ref[0]
    na = na_ref[0]
    relu = jax.nn.relu

    d = _dist_cols(nb_t_ref[0], p)
    s_sel = _top3_weights_cols(d)

    ii = ii_ref[0]
    iota = jax.lax.broadcasted_iota(jnp.int32, (256, p.shape[1]), 0)
    ohs, ds = [], []
    for k in range(3):
        oh = (iota == ii[k:k + 1, :]).astype(_F32)
        coords = _dot(na, oh)
        diff = p - coords
        ds.append(jnp.sqrt(jnp.sum(diff * diff, axis=0, keepdims=True)))
        ohs.append(oh)
    rs = 1.0 / (ds[0] + ds[1] + ds[2])
    s_a = (1.0 - ds[0] * rs) * ohs[0]
    for k in range(1, 3):
        s_a = s_a + (1.0 - ds[k] * rs) * ohs[k]

    h = relu(_dot(mb_ref[0], s_sel) + _dot(ma_ref[0], s_a)
             + _dot(P1_ref[:, 640:672], f1_ref[0])
             + _dot(P1_ref[:, 672:], f2_ref[0])
             + p1_ref[...])
    h = relu(_dot(P2_ref[...], h) + p2_ref[...])
    o = _dot(P3_ref[...], h) + p3_ref[...]
    coarse_ref[0] = o[0:2, :]
    fine_ref[0] = o[2:82, :]


def kernel(pc, node_a, node_b, first_pn_out, second_pn_out, node_a_features,
           node_b_features, global_feature, img_s16_feature_map,
           img_s32_feature_map, img_global_feature, params, node_a_min_k_idx):
    B, N = pc.shape[0], pc.shape[2]
    Ma, Mb = node_a.shape[2], node_b.shape[2]
    f32 = _F32

    s16 = img_s16_feature_map.reshape(B, img_s16_feature_map.shape[1], -1)
    s32 = img_s32_feature_map.reshape(B, img_s32_feature_map.shape[1], -1)
    ig = img_global_feature.reshape(B, img_global_feature.shape[1], 1)
    nb_t = node_b.transpose(0, 2, 1)
    ii_t = node_a_min_k_idx.astype(jnp.int32).transpose(0, 2, 1)

    col = lambda b: b.reshape(-1, 1)

    (W1, b1), (W2, b2) = params['nb_att']
    (V1, c1), (V2, c2), (V3, c3) = params['nb_pn']
    (A1, a1), (A2, a2) = params['na_att']
    (U1, u1), (U2, u2), (U3, u3) = params['na_pn']
    (P1, q1), (P2, q2), (P3, q3) = params['pp_pn']

    w_s1 = [W1, col(b1), W2, col(b2),
            V1, col(c1), V2, col(c2), V3, col(c3),
            A1, col(a1), A2, col(a2),
            U1, col(u1), U2, col(u2), U3, col(u3),
            P1]

    bspec = lambda *s: pl.BlockSpec((1,) + s, lambda b: (b, 0, 0))
    wspec = lambda w: pl.BlockSpec(w.shape, lambda b: (0,) * w.ndim)

    mb, ma = pl.pallas_call(
        _stage1_kernel,
        grid=(B,),
        in_specs=[bspec(256, Mb), bspec(64, Ma), bspec(512, 1), bspec(512, 1),
                  bspec(256, 320), bspec(512, 80), bspec(3, Ma), bspec(Mb, 3)]
                 + [wspec(w) for w in w_s1],
        out_specs=[bspec(256, Mb), bspec(256, Ma)],
        out_shape=[jax.ShapeDtypeStruct((B, 256, Mb), f32),
                   jax.ShapeDtypeStruct((B, 256, Ma), f32)],
    )(node_b_features, node_a_features, global_feature, ig, s16, s32,
      node_a, nb_t, *w_s1)

    w_s2 = [P1, col(q1), P2, col(q2), P3, col(q3)]

    bspec2 = lambda *s: pl.BlockSpec((1,) + s, lambda b, i: (b, 0, i))
    rep2 = lambda *s: pl.BlockSpec((1,) + s, lambda b, i: (b, 0, 0))
    wspec2 = lambda w: pl.BlockSpec(w.shape, lambda b, i: (0,) * w.ndim)

    coarse, fine = pl.pallas_call(
        _stage2_kernel,
        grid=(B, N // _NB),
        in_specs=[bspec2(3, _NB), bspec2(3, _NB), bspec2(32, _NB),
                  bspec2(64, _NB), rep2(Mb, 3), rep2(3, Ma),
                  rep2(256, Mb), rep2(256, Ma)]
                 + [wspec2(w) for w in w_s2],
        out_specs=[pl.BlockSpec((1, 2, _NB), lambda b, i: (b, 0, i)),
                   pl.BlockSpec((1, 80, _NB), lambda b, i: (b, 0, i))],
        out_shape=[jax.ShapeDtypeStruct((B, 2, N), f32),
                   jax.ShapeDtypeStruct((B, 80, N), f32)],
        compiler_params=pltpu.CompilerParams(
            dimension_semantics=("parallel", "parallel")),
    )(pc, ii_t, first_pn_out, second_pn_out, nb_t, node_a, mb, ma, *w_s2)

    return (coarse, fine)
